# baseline (speedup 1.0000x reference)
import functools

import jax
import jax.numpy as jnp
from jax import lax
from jax.experimental import pallas as pl
from jax.experimental.pallas import tpu as pltpu
from jax.experimental.pallas import tpu_sc as plsc

_LANES = 16
_UNIT = 1024
_NTILES = 32


def _sc_splat_body(H, W, N, n_iters,
                   cam_hbm, pts_hbm, zeros_hbm, out_hbm,
                   cam_v, pts_v, eidx_v, v0, v1, v2, v3, v4,
                   a0, a1, a2, a3, a4):
    HW = H * W
    c = lax.axis_index("c")
    s = lax.axis_index("s")
    wid = c * 16 + s
    planes = (a0, a1, a2, a3, a4)
    vals = (v0, v1, v2, v3, v4)

    rpt = HW // 16
    for p in planes:
        pltpu.sync_copy(zeros_hbm.at[pl.ds(s * rpt, rpt)],
                        p.at[pl.ds(s * rpt, rpt)])
    pltpu.sync_copy(cam_hbm, cam_v)
    plsc.subcore_barrier()

    fx = cam_v[pl.ds(0, 16)]
    fy = cam_v[pl.ds(16, 16)]
    cx = cam_v[pl.ds(32, 16)]
    cy = cam_v[pl.ds(48, 16)]
    tx = cam_v[pl.ds(64, 16)]
    ty = cam_v[pl.ds(80, 16)]
    tz = cam_v[pl.ds(96, 16)]
    iota = lax.iota(jnp.int32, 16)

    @pl.loop(0, n_iters)
    def _unit(k):
        for b in range(2):
            q = wid + _NTILES * (2 * k + b)
            pltpu.sync_copy(pts_hbm.at[:, pl.ds(q * _UNIT, _UNIT)], pts_v)
            if True:
                gbase = q * _UNIT
                for g in range(_UNIT // _LANES):

                    def col(ci):
                        return pts_v[ci, pl.ds(g * _LANES, _LANES)]

                    x, y, z = col(0), col(1), col(2)
                    pr, pg, pb = col(3), col(4), col(5)
                    pz = z + tz
                    zc = jnp.maximum(pz, 1e-3)
                    inv = 1.0 / zc
                    uu = jnp.clip(fx * (x + tx) / zc + cx, -4.0, 516.0)
                    vv = jnp.clip(fy * (y + ty) / zc + cy, -4.0, 516.0)

                    def rne(t):
                        ti = t.astype(jnp.int32)
                        fl = ti - jnp.where(t < ti.astype(jnp.float32), 1, 0)
                        d = t - fl.astype(jnp.float32)
                        up = (d > 0.5) | ((d == 0.5) & ((fl & 1) == 1))
                        return fl + jnp.where(up, 1, 0)

                    ui = rne(uu)
                    vi = rne(vv)
                    valid = ((pz > 1e-3) & (ui >= 0) & (ui < W)
                             & (vi >= 0) & (vi < H)
                             & (gbase + (g * _LANES) + iota < N))
                    w = jnp.where(valid, inv, 0.0)
                    uic = jnp.clip(ui, 0, W - 1)
                    vic = jnp.clip(vi, 0, H - 1)
                    pidx = vic * W + uic

                    sl = pl.ds(g * _LANES, _LANES)
                    eidx_v[sl] = pidx
                    v0[sl] = w * pr
                    v1[sl] = w * pg
                    v2[sl] = w * pb
                    v3[sl] = w * pz
                    v4[sl] = w
                for p, v in zip(planes, vals):
                    pltpu.sync_copy(v, p.at[eidx_v], add=True)

    plsc.subcore_barrier()
    for pi, p in enumerate(planes):
        out_off = c * (5 * HW) + pi * HW + s * rpt
        pltpu.sync_copy(p.at[pl.ds(s * rpt, rpt)],
                        out_hbm.at[pl.ds(out_off, rpt)])


def _tc_combine_body(p_ref, env_ref, out_ref):
    p = p_ref[...]
    acc = p[0, :4, :] + p[1, :4, :]
    ws = p[0, 4:, :] + p[1, 4:, :]
    out_ref[...] = jnp.where(ws > 0.0, acc / jnp.maximum(ws, 1e-6),
                             env_ref[...])


def kernel(cam_type, camera, points, environment, environment_type):
    H, W, C = environment.shape
    HW = H * W
    N = points.shape[0]

    n_units = -(-(-(-N // _UNIT)) // (2 * _NTILES)) * (2 * _NTILES)
    n_iters = n_units // (2 * _NTILES)
    npad = n_units * _UNIT - N
    pts = jnp.pad(points, ((0, npad), (0, 0))).T
    cam16 = jnp.repeat(camera[:, None], _LANES, axis=1).reshape(-1)
    zeros = jnp.zeros((HW,), jnp.float32)

    splat = pl.kernel(
        functools.partial(_sc_splat_body, H, W, N, n_iters),
        out_type=jax.ShapeDtypeStruct((2 * 5 * HW,), jnp.float32),
        mesh=plsc.VectorSubcoreMesh(core_axis_name="c",
                                    subcore_axis_name="s"),
        scratch_types=[
            pltpu.VMEM((7 * _LANES,), jnp.float32),
            pltpu.VMEM((6, _UNIT), jnp.float32),
            pltpu.VMEM((_UNIT,), jnp.int32),
        ] + [pltpu.VMEM((_UNIT,), jnp.float32)] * 5
          + [pltpu.VMEM_SHARED((HW,), jnp.float32)] * 5,
        compiler_params=pltpu.CompilerParams(needs_layout_passes=False),
    )
    partials = splat(cam16, pts, zeros).reshape(2, 5, HW)

    R = 16384
    combine = pl.pallas_call(
        _tc_combine_body,
        grid=(HW // R,),
        in_specs=[
            pl.BlockSpec((2, 5, R), lambda i: (0, 0, i)),
            pl.BlockSpec((C, R), lambda i: (0, i)),
        ],
        out_specs=pl.BlockSpec((C, R), lambda i: (0, i)),
        out_shape=jax.ShapeDtypeStruct((C, HW), jnp.float32),
    )
    env_t = environment.reshape(HW, C).T
    out = combine(partials, env_t)
    return out.T.reshape(H, W, C)


# speedup vs baseline: 1.8172x; 1.8172x over previous
import functools

import jax
import jax.numpy as jnp
from jax import lax
from jax.experimental import pallas as pl
from jax.experimental.pallas import tpu as pltpu
from jax.experimental.pallas import tpu_sc as plsc

_LANES = 16
_UNIT = 512
_NTILES = 32


def _sc_splat_body(H, W, N, n_iters,
                   cam_hbm, pts_hbm, zeros_hbm, out_hbm,
                   cam_v, pts_a, pts_b, eidx_a, eidx_b,
                   va0, va1, va2, va3, va4, vb0, vb1, vb2, vb3, vb4,
                   a0, a1, a2, a3, a4,
                   sem_la, sem_lb, sem_sa, sem_sb):
    HW = H * W
    c = lax.axis_index("c")
    s = lax.axis_index("s")
    wid = c * 16 + s
    planes = (a0, a1, a2, a3, a4)
    bufs = (
        (pts_a, eidx_a, (va0, va1, va2, va3, va4), sem_la, sem_sa),
        (pts_b, eidx_b, (vb0, vb1, vb2, vb3, vb4), sem_lb, sem_sb),
    )

    rpt = HW // 16
    for p in planes:
        pltpu.sync_copy(zeros_hbm.at[pl.ds(s * rpt, rpt)],
                        p.at[pl.ds(s * rpt, rpt)])
    pltpu.sync_copy(cam_hbm, cam_v)
    plsc.subcore_barrier()

    fx = cam_v[pl.ds(0, 16)]
    fy = cam_v[pl.ds(16, 16)]
    cx = cam_v[pl.ds(32, 16)]
    cy = cam_v[pl.ds(48, 16)]
    tx = cam_v[pl.ds(64, 16)]
    ty = cam_v[pl.ds(80, 16)]
    tz = cam_v[pl.ds(96, 16)]
    iota = lax.iota(jnp.int32, 16)

    def load_bytes_wait(sem, pts_v):
        pltpu.make_async_copy(pts_hbm.at[:, pl.ds(0, _UNIT)], pts_v,
                              sem).wait()

    def scatter_bytes_wait(sem, vals):
        for v in vals:
            pltpu.make_async_copy(zeros_hbm.at[pl.ds(0, _UNIT)], v,
                                  sem).wait()

    pltpu.async_copy(pts_hbm.at[:, pl.ds(wid * _UNIT, _UNIT)], pts_a, sem_la)

    @pl.loop(0, n_iters)
    def _unit(k):
        for b in range(2):
            pts_v, eidx_v, vals, sem_l, sem_s = bufs[b]
            o_pts, _, _, o_sem_l, _ = bufs[1 - b]
            q = wid + _NTILES * (2 * k + b)
            qn = q + _NTILES

            def start_next():
                pltpu.async_copy(
                    pts_hbm.at[:, pl.ds(qn * _UNIT, _UNIT)], o_pts, o_sem_l)

            if b == 0:
                start_next()
            else:
                @pl.when(k < n_iters - 1)
                def _guarded():
                    start_next()

            @pl.when(k > 0)
            def _drain():
                scatter_bytes_wait(sem_s, vals)

            load_bytes_wait(sem_l, pts_v)

            gbase = q * _UNIT
            for g in range(_UNIT // _LANES):

                def col(ci):
                    return pts_v[ci, pl.ds(g * _LANES, _LANES)]

                x, y, z = col(0), col(1), col(2)
                pr, pg, pb = col(3), col(4), col(5)
                pz = z + tz
                zc = jnp.maximum(pz, 1e-3)
                inv = 1.0 / zc
                uu = jnp.clip(fx * (x + tx) / zc + cx, -4.0, 516.0)
                vv = jnp.clip(fy * (y + ty) / zc + cy, -4.0, 516.0)

                def rne(t):
                    ti = t.astype(jnp.int32)
                    fl = ti - jnp.where(t < ti.astype(jnp.float32), 1, 0)
                    d = t - fl.astype(jnp.float32)
                    up = (d > 0.5) | ((d == 0.5) & ((fl & 1) == 1))
                    return fl + jnp.where(up, 1, 0)

                ui = rne(uu)
                vi = rne(vv)
                valid = ((pz > 1e-3) & (ui >= 0) & (ui < W)
                         & (vi >= 0) & (vi < H)
                         & (gbase + (g * _LANES) + iota < N))
                w = jnp.where(valid, inv, 0.0)
                uic = jnp.clip(ui, 0, W - 1)
                vic = jnp.clip(vi, 0, H - 1)
                pidx = vic * W + uic

                sl = pl.ds(g * _LANES, _LANES)
                eidx_v[sl] = pidx
                vals[0][sl] = w * pr
                vals[1][sl] = w * pg
                vals[2][sl] = w * pb
                vals[3][sl] = w * pz
                vals[4][sl] = w
            for p, v in zip(planes, vals):
                pltpu.async_copy(v, p.at[eidx_v], sem_s, add=True)

    for b in range(2):
        _, _, vals, _, sem_s = bufs[b]
        scatter_bytes_wait(sem_s, vals)

    plsc.subcore_barrier()
    for pi, p in enumerate(planes):
        out_off = c * (5 * HW) + pi * HW + s * rpt
        pltpu.sync_copy(p.at[pl.ds(s * rpt, rpt)],
                        out_hbm.at[pl.ds(out_off, rpt)])


def _tc_combine_body(p_ref, env_ref, out_ref):
    p = p_ref[...]
    acc = p[0, :4, :] + p[1, :4, :]
    ws = p[0, 4:, :] + p[1, 4:, :]
    out_ref[...] = jnp.where(ws > 0.0, acc / jnp.maximum(ws, 1e-6),
                             env_ref[...])


def kernel(cam_type, camera, points, environment, environment_type):
    H, W, C = environment.shape
    HW = H * W
    N = points.shape[0]

    n_units = -(-(-(-N // _UNIT)) // (2 * _NTILES)) * (2 * _NTILES)
    n_iters = n_units // (2 * _NTILES)
    npad = n_units * _UNIT - N
    pts = jnp.pad(points, ((0, npad), (0, 0))).T
    cam16 = jnp.repeat(camera[:, None], _LANES, axis=1).reshape(-1)
    zeros = jnp.zeros((HW,), jnp.float32)

    splat = pl.kernel(
        functools.partial(_sc_splat_body, H, W, N, n_iters),
        out_type=jax.ShapeDtypeStruct((2 * 5 * HW,), jnp.float32),
        mesh=plsc.VectorSubcoreMesh(core_axis_name="c",
                                    subcore_axis_name="s"),
        scratch_types=[
            pltpu.VMEM((7 * _LANES,), jnp.float32),
        ] + [pltpu.VMEM((6, _UNIT), jnp.float32)] * 2
          + [pltpu.VMEM((_UNIT,), jnp.int32)] * 2
          + [pltpu.VMEM((_UNIT,), jnp.float32)] * 10
          + [pltpu.VMEM_SHARED((HW,), jnp.float32)] * 5
          + [pltpu.SemaphoreType.DMA] * 4,
        compiler_params=pltpu.CompilerParams(needs_layout_passes=False),
    )
    partials = splat(cam16, pts, zeros).reshape(2, 5, HW)

    R = 16384
    combine = pl.pallas_call(
        _tc_combine_body,
        grid=(HW // R,),
        in_specs=[
            pl.BlockSpec((2, 5, R), lambda i: (0, 0, i)),
            pl.BlockSpec((C, R), lambda i: (0, i)),
        ],
        out_specs=pl.BlockSpec((C, R), lambda i: (0, i)),
        out_shape=jax.ShapeDtypeStruct((C, HW), jnp.float32),
    )
    env_t = environment.reshape(HW, C).T
    out = combine(partials, env_t)
    return out.T.reshape(H, W, C)
